# baseline (speedup 1.0000x reference)
import functools

import jax
import jax.numpy as jnp
from jax import lax
from jax.experimental import pallas as pl
from jax.experimental.pallas import tpu as pltpu
from jax.experimental.pallas import tpu_sc as plsc

VOCAB = 16
NFRAME = 3
D = 1024
FRAG = D // 128
NC = 2
NS = 16
NW = NC * NS
LANES = 16
PIECES = D // LANES
CHUNK = 32


@functools.partial(jax.jit, static_argnames=("n_batch", "seq"))
def _run(ids, fp_pad, word1, frame1, n_batch, seq):
    n_rows = n_batch * seq
    rows_per_w = n_rows // NW
    n_chunks = rows_per_w // CHUNK
    workers_per_batch = NW // n_batch
    mesh = plsc.VectorSubcoreMesh(
        core_axis_name="c", subcore_axis_name="s",
        num_cores=NC, num_subcores=NS)

    @functools.partial(
        pl.kernel,
        out_type=jax.ShapeDtypeStruct((n_batch, seq, D), jnp.float32),
        mesh=mesh,
        scratch_types=[
            pltpu.VMEM((NFRAME * VOCAB * D,), jnp.float32),
            pltpu.VMEM((NFRAME * D,), jnp.float32),
            pltpu.VMEM((2, CHUNK, D), jnp.float32),
            pltpu.VMEM((2 * LANES,), jnp.int32),
            pltpu.VMEM((rows_per_w,), jnp.int32),
            pltpu.SemaphoreType.DMA,
        ],
    )
    def k(ids_hbm, fp_hbm, word1_hbm, frame1_hbm, out_hbm,
          comb_v, ftab_v, rows_v, fp_v, ids_v, ssem):
        cid = lax.axis_index("c")
        sid = lax.axis_index("s")
        wid = cid * NS + sid
        b = wid // workers_per_batch
        l_base = pl.multiple_of(
            (wid % workers_per_batch) * rows_per_w, rows_per_w)

        for m in range(NFRAME):
            pltpu.sync_copy(
                word1_hbm, comb_v.at[pl.ds(m * VOCAB * D, VOCAB * D)])
        pltpu.sync_copy(frame1_hbm, ftab_v)
        pltpu.sync_copy(ids_hbm.at[b, pl.ds(l_base, rows_per_w)], ids_v)
        pltpu.sync_copy(fp_hbm, fp_v)

        def build(g, carry):
            for m in range(NFRAME):
                for cb in range(FRAG):
                    sl = pl.ds(m * D + cb * 128 + g * LANES, LANES)
                    f = ftab_v[sl]
                    for v in range(VOCAB):
                        dst = pl.ds(
                            (m * VOCAB + v) * D + cb * 128 + g * LANES, LANES)
                        comb_v[dst] = comb_v[dst] + f
            return carry

        lax.fori_loop(0, 128 // LANES, build, 0)

        fp_vec = fp_v[pl.ds(0, LANES)]
        phase = fp_vec[0]
        for j in range(1, n_batch):
            phase = jnp.where(b == j, fp_vec[j], phase)

        def out_slice(c):
            return out_hbm.at[
                b, pl.ds(pl.multiple_of(l_base + c * CHUNK, CHUNK), CHUNK)]

        def scatter_wait(c, buf):
            pltpu.make_async_copy(rows_v.at[buf], out_slice(c), ssem).wait()

        def chunk_loop(c, carry):
            buf = c % 2

            @pl.when(c >= 2)
            def _():
                scatter_wait(c - 2, buf)

            for i16 in range(CHUNK // LANES):
                off = c * CHUNK + i16 * LANES
                tok = ids_v[pl.ds(off, LANES)]
                pos = l_base + off + lax.iota(jnp.int32, LANES)
                cbase = (((phase + pos) % NFRAME) * VOCAB + tok) * D
                for j in range(0, LANES, 2):
                    src0 = cbase[j]
                    src1 = cbase[j + 1]
                    r0 = i16 * LANES + j

                    @plsc.parallel_loop(0, PIECES, 1, unroll=4)
                    def piece(p):
                        sl = pl.ds(p * LANES, LANES)
                        rows_v[buf, r0, sl] = comb_v[pl.ds(
                            src0 + p * LANES, LANES)]
                        rows_v[buf, r0 + 1, sl] = comb_v[pl.ds(
                            src1 + p * LANES, LANES)]
            pltpu.async_copy(rows_v.at[buf], out_slice(c), ssem)
            return carry

        lax.fori_loop(0, n_chunks, chunk_loop, 0)
        scatter_wait(n_chunks - 2, n_chunks % 2)
        scatter_wait(n_chunks - 1, 1 - n_chunks % 2)

    return k(ids, fp_pad, word1, frame1)


def kernel(ids, frame_phase, word_emb, frame_emb):
    n_batch, seq = ids.shape
    ids32 = ids.astype(jnp.int32)
    fp_pad = jnp.zeros((2 * LANES,), jnp.int32).at[:n_batch].set(
        frame_phase.astype(jnp.int32))
    word1 = word_emb.reshape(VOCAB * D)
    frame1 = frame_emb.reshape(NFRAME * D)
    return _run(ids32, fp_pad, word1, frame1, n_batch, seq)


# speedup vs baseline: 1.0256x; 1.0256x over previous
import functools

import jax
import jax.numpy as jnp
from jax import lax
from jax.experimental import pallas as pl
from jax.experimental.pallas import tpu as pltpu
from jax.experimental.pallas import tpu_sc as plsc

VOCAB = 16
NFRAME = 3
D = 1024
FRAG = D // 128
NC = 2
NS = 16
NW = NC * NS
LANES = 16
PIECES = D // LANES
CHUNK = 32


@functools.partial(jax.jit, static_argnames=("n_batch", "seq"))
def _run(ids, fp_pad, word1, frame1, n_batch, seq):
    n_rows = n_batch * seq
    rows_per_w = n_rows // NW
    n_chunks = rows_per_w // CHUNK
    workers_per_batch = NW // n_batch
    mesh = plsc.VectorSubcoreMesh(
        core_axis_name="c", subcore_axis_name="s",
        num_cores=NC, num_subcores=NS)

    @functools.partial(
        pl.kernel,
        out_type=jax.ShapeDtypeStruct((n_batch, seq, D), jnp.float32),
        mesh=mesh,
        scratch_types=[
            pltpu.VMEM((NFRAME * VOCAB * D,), jnp.float32),
            pltpu.VMEM((NFRAME * D,), jnp.float32),
            pltpu.VMEM((2, CHUNK, D), jnp.float32),
            pltpu.VMEM((2 * LANES,), jnp.int32),
            pltpu.VMEM((rows_per_w,), jnp.int32),
            pltpu.SemaphoreType.DMA,
        ],
    )
    def k(ids_hbm, fp_hbm, word1_hbm, frame1_hbm, out_hbm,
          comb_v, ftab_v, rows_v, fp_v, ids_v, ssem):
        cid = lax.axis_index("c")
        sid = lax.axis_index("s")
        wid = cid * NS + sid
        b = wid // workers_per_batch
        l_base = pl.multiple_of(
            (wid % workers_per_batch) * rows_per_w, rows_per_w)

        for m in range(NFRAME):
            pltpu.sync_copy(
                word1_hbm, comb_v.at[pl.ds(m * VOCAB * D, VOCAB * D)])
        pltpu.sync_copy(frame1_hbm, ftab_v)
        pltpu.sync_copy(ids_hbm.at[b, pl.ds(l_base, rows_per_w)], ids_v)
        pltpu.sync_copy(fp_hbm, fp_v)

        def build(g, carry):
            for m in range(NFRAME):
                for cb in range(FRAG):
                    sl = pl.ds(m * D + cb * 128 + g * LANES, LANES)
                    f = ftab_v[sl]
                    for v in range(VOCAB):
                        dst = pl.ds(
                            (m * VOCAB + v) * D + cb * 128 + g * LANES, LANES)
                        comb_v[dst] = comb_v[dst] + f
            return carry

        lax.fori_loop(0, 128 // LANES, build, 0)

        fp_vec = fp_v[pl.ds(0, LANES)]
        phase = fp_vec[0]
        for j in range(1, n_batch):
            phase = jnp.where(b == j, fp_vec[j], phase)

        def out_slice(c):
            return out_hbm.at[
                b, pl.ds(pl.multiple_of(l_base + c * CHUNK, CHUNK), CHUNK)]

        def scatter_wait(c, buf):
            pltpu.make_async_copy(rows_v.at[buf], out_slice(c), ssem).wait()

        def chunk_loop(c, carry):
            buf = c % 2

            @pl.when(c >= 2)
            def _():
                scatter_wait(c - 2, buf)

            for i16 in range(CHUNK // LANES):
                off = c * CHUNK + i16 * LANES
                tok = ids_v[pl.ds(off, LANES)]
                pos = l_base + off + lax.iota(jnp.int32, LANES)
                cbase = (((phase + pos) % NFRAME) * VOCAB + tok) * D
                for j in range(LANES):
                    src = cbase[j]
                    r = i16 * LANES + j

                    @plsc.parallel_loop(0, PIECES, 1, unroll=8)
                    def piece(p):
                        rows_v[buf, r, pl.ds(p * LANES, LANES)] = (
                            comb_v[pl.ds(src + p * LANES, LANES)])
            pltpu.async_copy(rows_v.at[buf], out_slice(c), ssem)
            return carry

        lax.fori_loop(0, n_chunks, chunk_loop, 0)
        scatter_wait(n_chunks - 2, n_chunks % 2)
        scatter_wait(n_chunks - 1, 1 - n_chunks % 2)

    return k(ids, fp_pad, word1, frame1)


def kernel(ids, frame_phase, word_emb, frame_emb):
    n_batch, seq = ids.shape
    ids32 = ids.astype(jnp.int32)
    fp_pad = jnp.zeros((2 * LANES,), jnp.int32).at[:n_batch].set(
        frame_phase.astype(jnp.int32))
    word1 = word_emb.reshape(VOCAB * D)
    frame1 = frame_emb.reshape(NFRAME * D)
    return _run(ids32, fp_pad, word1, frame1, n_batch, seq)
